# baseline (speedup 1.0000x reference)
import jax
import jax.numpy as jnp
from jax import lax
from jax.experimental import pallas as pl
from jax.experimental.pallas import tpu as pltpu
from jax.experimental.pallas import tpu_sc as plsc

N = 2_000_000
G = 256
GN = G * G * G
NW = 32
PT = N // NW
CH = 4992
NCH = 13
NS = 4
SS = CH // NS
SGRP = SS // 16


def _occupancy_body(x_ref, y_ref, z_ref, grid_ref,
                    xb0, yb0, zb0, xb1, yb1, zb1, idx0, idx1, ones_v,
                    psem0, psem1, ssem0, ssem1):
    wid = lax.axis_index("c") * 16 + lax.axis_index("s")
    odd = wid % 2
    base = PT * wid - 4 * odd
    size = 62496 + 8 * odd

    @plsc.parallel_loop(0, SGRP, unroll=8)
    def _ofill(i):
        ones_v[pl.ds(i * 16, 16)] = jnp.ones((16,), jnp.int32)

    pts_bufs = ((xb0, yb0, zb0), (xb1, yb1, zb1))
    idx_bufs = (idx0, idx1)
    psems = (psem0, psem1)
    ssems = (ssem0, ssem1)

    def _chunk_start(c):
        if c < NCH - 1:
            return pl.multiple_of(base + c * CH, 8)
        return pl.multiple_of(base + size - CH, 8)

    def _start_load(c):
        b = pts_bufs[c % 2]
        sem = psems[c % 2]
        sl = pl.ds(_chunk_start(c), CH)
        return (pltpu.async_copy(x_ref.at[sl], b[0], sem),
                pltpu.async_copy(y_ref.at[sl], b[1], sem),
                pltpu.async_copy(z_ref.at[sl], b[2], sem))

    pload = [None] * NCH
    pload[0] = _start_load(0)
    pload[1] = _start_load(1)

    def _compute(bufs, idx):
        xb, yb, zb = bufs
        for k in range(NS):
            seg = idx[k]

            @plsc.parallel_loop(0, SGRP, unroll=8)
            def _grp(g):
                row = pl.ds(k * SS + g * 16, 16)
                fx = xb[row] * 256.0
                fy = yb[row] * 256.0
                fz = zb[row] * 256.0
                inb = (fx <= 255.0) & (fy <= 255.0) & (fz <= 255.0)
                v = ((fx.astype(jnp.int32) << 16)
                     | (fy.astype(jnp.int32) << 8)
                     | fz.astype(jnp.int32))
                seg[pl.ds(g * 16, 16)] = jnp.where(inb, v, -1)

    scat = [None] * NCH
    for c in range(NCH):
        b = c % 2
        for cp in pload[c]:
            cp.wait()
        if c >= 2:
            for s in scat[c - 2]:
                s.wait()
        _compute(pts_bufs[b], idx_bufs[b])
        scat[c] = [
            pltpu.async_copy(
                ones_v,
                grid_ref.at[plsc.Indices(idx_bufs[b][k], ignored_value=-1)],
                ssems[b])
            for k in range(NS)
        ]
        if c + 2 < NCH:
            pload[c + 2] = _start_load(c + 2)
    for s in scat[NCH - 2] + scat[NCH - 1]:
        s.wait()


@jax.jit
def _occupancy(points):
    mesh = plsc.VectorSubcoreMesh(
        core_axis_name="c", subcore_axis_name="s", num_cores=2)

    def _stateful(refs):
        x_ref, y_ref, z_ref, grid_ref = refs

        @pl.core_map(
            mesh,
            compiler_params=pltpu.CompilerParams(needs_layout_passes=False),
            scratch_shapes=[
                pltpu.VMEM((CH,), jnp.float32),
                pltpu.VMEM((CH,), jnp.float32),
                pltpu.VMEM((CH,), jnp.float32),
                pltpu.VMEM((CH,), jnp.float32),
                pltpu.VMEM((CH,), jnp.float32),
                pltpu.VMEM((CH,), jnp.float32),
                [pltpu.VMEM((SS,), jnp.int32) for _ in range(NS)],
                [pltpu.VMEM((SS,), jnp.int32) for _ in range(NS)],
                pltpu.VMEM((SS,), jnp.int32),
                pltpu.SemaphoreType.DMA,
                pltpu.SemaphoreType.DMA,
                pltpu.SemaphoreType.DMA,
                pltpu.SemaphoreType.DMA,
            ],
        )
        def _(*scratch):
            _occupancy_body(x_ref, y_ref, z_ref, grid_ref, *scratch)

    grid0 = jnp.zeros((GN,), jnp.int32)
    _, _, _, grid32 = pl.run_state(_stateful)(
        (points[:, 0], points[:, 1], points[:, 2], grid0))
    return grid32.astype(jnp.bool_).reshape(G, G, G)


def kernel(points):
    return _occupancy(points)


# speedup vs baseline: 1.0387x; 1.0387x over previous
import jax
import jax.numpy as jnp
from jax import lax
from jax.experimental import pallas as pl
from jax.experimental.pallas import tpu as pltpu
from jax.experimental.pallas import tpu_sc as plsc

N = 2_000_000
G = 256
GN = G * G * G
NW = 16
PT = N // NW
CH = 4992
NCH = 26
TAIL = PT - (NCH - 1) * CH
NGRP = CH // 16


def _occupancy_body(x_ref, y_ref, z_ref, grid_ref,
                    xb0, yb0, zb0, xb1, yb1, zb1, idx0, idx1, ones_v,
                    psem0, psem1, ssem0, ssem1):
    wid = lax.axis_index("s")
    base = PT * wid

    @plsc.parallel_loop(0, NGRP, unroll=8)
    def _ofill(i):
        ones_v[pl.ds(i * 16, 16)] = jnp.ones((16,), jnp.int32)

    pts_bufs = ((xb0, yb0, zb0), (xb1, yb1, zb1))
    idx_bufs = (idx0, idx1)
    psems = (psem0, psem1)
    ssems = (ssem0, ssem1)

    def _chunk_start(c):
        if c < NCH - 1:
            return pl.multiple_of(base + c * CH, 8)
        return pl.multiple_of(base + PT - CH, 8)

    def _start_load(c):
        b = pts_bufs[c % 2]
        sem = psems[c % 2]
        sl = pl.ds(_chunk_start(c), CH)
        return (pltpu.async_copy(x_ref.at[sl], b[0], sem),
                pltpu.async_copy(y_ref.at[sl], b[1], sem),
                pltpu.async_copy(z_ref.at[sl], b[2], sem))

    pload = [None] * NCH
    pload[0] = _start_load(0)
    pload[1] = _start_load(1)

    lane = jnp.arange(16, dtype=jnp.int32)

    def _compute(bufs, idx, tail):
        xb, yb, zb = bufs

        @plsc.parallel_loop(0, NGRP, unroll=8)
        def _grp(g):
            row = pl.ds(g * 16, 16)
            fx = xb[row] * 256.0
            fy = yb[row] * 256.0
            fz = zb[row] * 256.0
            inb = (fx <= 255.0) & (fy <= 255.0) & (fz <= 255.0)
            if tail:
                inb = inb & (g * 16 + lane >= CH - TAIL)
            v = ((fx.astype(jnp.int32) << 16)
                 | (fy.astype(jnp.int32) << 8)
                 | fz.astype(jnp.int32))
            idx[row] = jnp.where(inb, v, -1)

    scat = [None] * NCH
    for c in range(NCH):
        b = c % 2
        for cp in pload[c]:
            cp.wait()
        if c >= 2:
            scat[c - 2].wait()
        _compute(pts_bufs[b], idx_bufs[b], c == NCH - 1)
        scat[c] = pltpu.async_copy(
            ones_v,
            grid_ref.at[plsc.Indices(idx_bufs[b], ignored_value=-1)],
            ssems[b])
        if c + 2 < NCH:
            pload[c + 2] = _start_load(c + 2)
    scat[NCH - 2].wait()
    scat[NCH - 1].wait()


@jax.jit
def _occupancy(points):
    mesh = plsc.VectorSubcoreMesh(
        core_axis_name="c", subcore_axis_name="s", num_cores=1)

    def _stateful(refs):
        x_ref, y_ref, z_ref, grid_ref = refs

        @pl.core_map(
            mesh,
            compiler_params=pltpu.CompilerParams(needs_layout_passes=False),
            scratch_shapes=[
                pltpu.VMEM((CH,), jnp.float32),
                pltpu.VMEM((CH,), jnp.float32),
                pltpu.VMEM((CH,), jnp.float32),
                pltpu.VMEM((CH,), jnp.float32),
                pltpu.VMEM((CH,), jnp.float32),
                pltpu.VMEM((CH,), jnp.float32),
                pltpu.VMEM((CH,), jnp.int32),
                pltpu.VMEM((CH,), jnp.int32),
                pltpu.VMEM((CH,), jnp.int32),
                pltpu.SemaphoreType.DMA,
                pltpu.SemaphoreType.DMA,
                pltpu.SemaphoreType.DMA,
                pltpu.SemaphoreType.DMA,
            ],
        )
        def _(*scratch):
            _occupancy_body(x_ref, y_ref, z_ref, grid_ref, *scratch)

    grid0 = jnp.zeros((GN,), jnp.int32)
    _, _, _, grid32 = pl.run_state(_stateful)(
        (points[:, 0], points[:, 1], points[:, 2], grid0))
    return grid32.astype(jnp.bool_).reshape(G, G, G)


def kernel(points):
    return _occupancy(points)
